# baseline (speedup 1.0000x reference)
import jax
import jax.numpy as jnp
from jax.experimental import pallas as pl
from jax.experimental.pallas import tpu as pltpu

_B, _V, _D, _P = 64, 321, 128, 12
_DP = _D * _P
_TW = 96

_VB = 24
_PB = 6


def _head_kernel(xt0_ref, xt1_ref, xf0_ref, xf1_ref, w1_ref, w2_ref, b_ref,
                 o_ref):
    j = pl.program_id(1)
    accs = []
    for xt_ref, xf_ref in ((xt0_ref, xf0_ref), (xt1_ref, xf1_ref)):
        for vi in range(_VB):
            acc = jnp.zeros((_B, _TW), jnp.float32)
            for p in range(_PB):
                acc = acc + jnp.dot(xt_ref[vi, p], w1_ref[p],
                                    preferred_element_type=jnp.float32)
                acc = acc + jnp.dot(xf_ref[vi, p], w2_ref[p],
                                    preferred_element_type=jnp.float32)
            accs.append(acc)
    part = jnp.stack(accs, axis=0).transpose(1, 0, 2)

    @pl.when(j == 0)
    def _init():
        o_ref[...] = part + b_ref[...].reshape(1, 1, _TW)

    @pl.when(j != 0)
    def _accum():
        o_ref[...] = o_ref[...] + part


def kernel(x_time, x_frequency, W, b):
    xt = x_time.transpose(1, 3, 0, 2)
    xf = x_frequency.transpose(1, 3, 0, 2)
    w1 = W[:, :_DP].reshape(_TW, _D, _P).transpose(2, 1, 0)
    w2 = W[:, _DP:].reshape(_TW, _D, _P).transpose(2, 1, 0)
    b2 = b.reshape(1, _TW)
    x_spec0 = pl.BlockSpec((_VB, _PB, _B, _D), lambda i, j: (2 * i, j, 0, 0))
    x_spec1 = pl.BlockSpec((_VB, _PB, _B, _D),
                           lambda i, j: (2 * i + 1, j, 0, 0))
    w_spec = pl.BlockSpec((_PB, _D, _TW), lambda i, j: (j, 0, 0))
    out = pl.pallas_call(
        _head_kernel,
        grid=(pl.cdiv(_V, 2 * _VB), _P // _PB),
        in_specs=[
            x_spec0, x_spec1, x_spec0, x_spec1,
            w_spec, w_spec,
            pl.BlockSpec((1, _TW), lambda i, j: (0, 0)),
        ],
        out_specs=pl.BlockSpec((_B, 2 * _VB, _TW), lambda i, j: (0, i, 0)),
        out_shape=jax.ShapeDtypeStruct((_B, _V, _TW), jnp.float32),
        compiler_params=pltpu.CompilerParams(
            dimension_semantics=("parallel", "arbitrary"),
        ),
    )(xt, xt, xf, xf, w1, w2, b2)
    return out


# speedup vs baseline: 1.0291x; 1.0291x over previous
import jax
import jax.numpy as jnp
from jax.experimental import pallas as pl
from jax.experimental.pallas import tpu as pltpu

_B, _V, _D, _P = 64, 321, 128, 12
_DP = _D * _P
_TW = 96

_VB = 24


def _head_kernel(xt_ref, xf_ref, w1_ref, w2_ref, b_ref, o_ref):
    accs = []
    for vi in range(_VB):
        acc = jnp.broadcast_to(b_ref[...], (_B, _TW)).astype(jnp.float32)
        for p in range(_P):
            acc = acc + jnp.dot(xt_ref[vi, p], w1_ref[p],
                                preferred_element_type=jnp.float32)
            acc = acc + jnp.dot(xf_ref[vi, p], w2_ref[p],
                                preferred_element_type=jnp.float32)
        accs.append(acc)
    o_ref[...] = jnp.stack(accs, axis=0).transpose(1, 0, 2)


def kernel(x_time, x_frequency, W, b):
    xt = x_time.transpose(1, 3, 0, 2)
    xf = x_frequency.transpose(1, 3, 0, 2)
    w1 = W[:, :_DP].reshape(_TW, _D, _P).transpose(2, 1, 0)
    w2 = W[:, _DP:].reshape(_TW, _D, _P).transpose(2, 1, 0)
    b2 = b.reshape(1, _TW)
    out = pl.pallas_call(
        _head_kernel,
        grid=(pl.cdiv(_V, _VB),),
        in_specs=[
            pl.BlockSpec((_VB, _P, _B, _D), lambda i: (i, 0, 0, 0)),
            pl.BlockSpec((_VB, _P, _B, _D), lambda i: (i, 0, 0, 0)),
            pl.BlockSpec((_P, _D, _TW), lambda i: (0, 0, 0)),
            pl.BlockSpec((_P, _D, _TW), lambda i: (0, 0, 0)),
            pl.BlockSpec((1, _TW), lambda i: (0, 0)),
        ],
        out_specs=pl.BlockSpec((_B, _VB, _TW), lambda i: (0, i, 0)),
        out_shape=jax.ShapeDtypeStruct((_B, _V, _TW), jnp.float32),
        compiler_params=pltpu.CompilerParams(
            dimension_semantics=("parallel",),
        ),
    )(xt, xf, w1, w2, b2)
    return out
